# speedup vs baseline: 1.0843x; 1.0843x over previous
import functools

import jax
import jax.numpy as jnp
from jax import lax
from jax.experimental import pallas as pl
from jax.experimental.pallas import tpu as pltpu
from jax.experimental.pallas import tpu_sc as plsc

NUM_CORES = 2
NUM_SUBCORES = 16
LANES = 16
NW = NUM_CORES * NUM_SUBCORES

BATCH = 16384
CHUNK = 128
BPW = BATCH // NW
ROWS_PW = BPW // CHUNK


@functools.partial(
    pl.kernel,
    mesh=plsc.VectorSubcoreMesh(core_axis_name="c", subcore_axis_name="s"),
    out_type=jax.ShapeDtypeStruct((BATCH // CHUNK, CHUNK), jnp.float32),
    scratch_types=[
        pltpu.VMEM((ROWS_PW, CHUNK), jnp.int32),
        pltpu.VMEM((ROWS_PW, CHUNK), jnp.int32),
        pltpu.VMEM((ROWS_PW, CHUNK), jnp.float32),
        pltpu.VMEM((ROWS_PW, CHUNK), jnp.float32),
        pltpu.VMEM((LANES,), jnp.float32),
        pltpu.SemaphoreType.DMA,
    ],
)
def _nbm_kernel(user_hbm, movie_hbm, ubias_hbm, mbias_hbm, gb_hbm, out_hbm,
                uidx, midx, uval, mval, gbv, sem):
    wid = lax.axis_index("s") * NUM_CORES + lax.axis_index("c")
    row0 = wid * ROWS_PW
    pltpu.sync_copy(user_hbm.at[pl.ds(row0, ROWS_PW)], uidx)
    pltpu.sync_copy(movie_hbm.at[pl.ds(row0, ROWS_PW)], midx)
    pltpu.sync_copy(gb_hbm, gbv)
    copies = []
    for j in range(ROWS_PW):
        copies.append(pltpu.async_copy(ubias_hbm.at[uidx.at[j]], uval.at[j], sem))
        copies.append(pltpu.async_copy(mbias_hbm.at[midx.at[j]], mval.at[j], sem))
    for c in copies:
        c.wait()
    g = gbv[...]
    for j in range(ROWS_PW):
        for i in range(CHUNK // LANES):
            sl = pl.ds(i * LANES, LANES)
            uval[j, sl] = uval[j, sl] + mval[j, sl] + g
    pltpu.sync_copy(uval, out_hbm.at[pl.ds(row0, ROWS_PW)])


def kernel(user, movie, user_biases, movie_biases, global_bias):
    user2d = user.reshape(BATCH // CHUNK, CHUNK)
    movie2d = movie.reshape(BATCH // CHUNK, CHUNK)
    ub = user_biases.reshape(-1)
    mb = movie_biases.reshape(-1)
    gb = jnp.broadcast_to(global_bias.reshape(1), (LANES,))
    out = _nbm_kernel(user2d, movie2d, ub, mb, gb)
    return out.reshape(BATCH)


# speedup vs baseline: 1.0970x; 1.0117x over previous
import functools

import jax
import jax.numpy as jnp
from jax import lax
from jax.experimental import pallas as pl
from jax.experimental.pallas import tpu as pltpu
from jax.experimental.pallas import tpu_sc as plsc

NUM_CORES = 2
NUM_SUBCORES = 16
LANES = 16
NW = NUM_CORES * NUM_SUBCORES

BATCH = 16384
CHUNK = 128
BPW = BATCH // NW
ROWS_PW = BPW // CHUNK


@functools.partial(
    pl.kernel,
    mesh=plsc.VectorSubcoreMesh(core_axis_name="c", subcore_axis_name="s"),
    out_type=jax.ShapeDtypeStruct((BATCH,), jnp.float32),
    scratch_types=[
        pltpu.VMEM((BPW,), jnp.int32),
        pltpu.VMEM((BPW,), jnp.int32),
        pltpu.VMEM((ROWS_PW, CHUNK), jnp.float32),
        pltpu.VMEM((ROWS_PW, CHUNK), jnp.float32),
        pltpu.VMEM((BPW,), jnp.float32),
        pltpu.VMEM((LANES,), jnp.float32),
        pltpu.SemaphoreType.DMA,
    ],
    compiler_params=pltpu.CompilerParams(use_tc_tiling_on_sc=False),
)
def _nbm_kernel(user_hbm, movie_hbm, ubias_hbm, mbias_hbm, gb_hbm, out_hbm,
                uidx, midx, uval, mval, outv, gbv, sem):
    wid = lax.axis_index("s") * NUM_CORES + lax.axis_index("c")
    base = wid * BPW
    pltpu.sync_copy(user_hbm.at[pl.ds(base, BPW)], uidx)
    pltpu.sync_copy(movie_hbm.at[pl.ds(base, BPW)], midx)
    pltpu.sync_copy(gb_hbm, gbv)
    copies = []
    for j in range(ROWS_PW):
        isl = pl.ds(j * CHUNK, CHUNK)
        copies.append(pltpu.async_copy(ubias_hbm.at[uidx.at[isl]], uval.at[j], sem))
        copies.append(pltpu.async_copy(mbias_hbm.at[midx.at[isl]], mval.at[j], sem))
    for c in copies:
        c.wait()
    g = gbv[...]
    for j in range(ROWS_PW):
        for i in range(CHUNK // LANES):
            sl = pl.ds(i * LANES, LANES)
            outv[pl.ds(j * CHUNK + i * LANES, LANES)] = (
                uval[j, sl] + mval[j, sl] + g)
    pltpu.sync_copy(outv, out_hbm.at[pl.ds(base, BPW)])


def kernel(user, movie, user_biases, movie_biases, global_bias):
    ub = user_biases.reshape(-1)
    mb = movie_biases.reshape(-1)
    gb = jnp.broadcast_to(global_bias.reshape(1), (LANES,))
    return _nbm_kernel(user, movie, ub, mb, gb)


# speedup vs baseline: 1.1857x; 1.0808x over previous
import functools

import jax
import jax.numpy as jnp
from jax import lax
from jax.experimental import pallas as pl
from jax.experimental.pallas import tpu as pltpu
from jax.experimental.pallas import tpu_sc as plsc

NUM_CORES = 2
NUM_SUBCORES = 16
LANES = 16
NW = NUM_CORES * NUM_SUBCORES

BATCH = 16384
CHUNK = 128
BPW = BATCH // NW
ROWS_PW = BPW // CHUNK


@functools.partial(
    pl.kernel,
    mesh=plsc.VectorSubcoreMesh(core_axis_name="c", subcore_axis_name="s"),
    out_type=jax.ShapeDtypeStruct((BATCH,), jnp.float32),
    scratch_types=[
        pltpu.VMEM((BPW,), jnp.int32),
        pltpu.VMEM((BPW,), jnp.int32),
        pltpu.VMEM((ROWS_PW, CHUNK), jnp.float32),
        pltpu.VMEM((ROWS_PW, CHUNK), jnp.float32),
        pltpu.VMEM((BPW,), jnp.float32),
        pltpu.VMEM((LANES,), jnp.float32),
        pltpu.SemaphoreType.DMA,
    ],
    compiler_params=pltpu.CompilerParams(use_tc_tiling_on_sc=False),
)
def _nbm_kernel(user_hbm, movie_hbm, ubias_hbm, mbias_hbm, gb_hbm, out_hbm,
                uidx, midx, uval, mval, outv, gbv, sem):
    ub1 = ubias_hbm.at[0]
    mb1 = mbias_hbm.at[0]
    wid = lax.axis_index("s") * NUM_CORES + lax.axis_index("c")
    base = wid * BPW
    pltpu.sync_copy(user_hbm.at[pl.ds(base, BPW)], uidx)
    pltpu.sync_copy(movie_hbm.at[pl.ds(base, BPW)], midx)
    pltpu.sync_copy(gb_hbm, gbv)
    copies = []
    for j in range(ROWS_PW):
        isl = pl.ds(j * CHUNK, CHUNK)
        copies.append(pltpu.async_copy(ub1.at[uidx.at[isl]], uval.at[j], sem))
        copies.append(pltpu.async_copy(mb1.at[midx.at[isl]], mval.at[j], sem))
    for c in copies:
        c.wait()
    g = gbv[...]
    for j in range(ROWS_PW):
        for i in range(CHUNK // LANES):
            sl = pl.ds(i * LANES, LANES)
            outv[pl.ds(j * CHUNK + i * LANES, LANES)] = (
                uval[j, sl] + mval[j, sl] + g)
    pltpu.sync_copy(outv, out_hbm.at[pl.ds(base, BPW)])


def kernel(user, movie, user_biases, movie_biases, global_bias):
    ub = user_biases.reshape(1, -1)
    mb = movie_biases.reshape(1, -1)
    gb = jnp.broadcast_to(global_bias.reshape(1), (LANES,))
    return _nbm_kernel(user, movie, ub, mb, gb)
